# Initial kernel scaffold; baseline (speedup 1.0000x reference)
import jax
import jax.numpy as jnp
from jax.experimental import pallas as pl


def kernel(x, edge_index, W_self0, b_self0, W_neigh0, b_neigh0, gamma0, beta0, W_self1, b_self1, W_neigh1, b_neigh1, gamma1, beta1, W_self2, b_self2, W_neigh2, b_neigh2):
    raise NotImplementedError("write your pallas kernel here")


# speedup vs baseline: 4.7336x; 4.7336x over previous
import functools

import jax
import jax.numpy as jnp
from jax import lax
from jax.experimental import pallas as pl
from jax.experimental.pallas import tpu as pltpu
from jax.experimental.pallas import tpu_sc as plsc

_EPS = 1e-5
_NC = 2
_NS = 16
_NW = _NC * _NS


def _chunk_size(epw, cap=128):
    for k in range(cap, 0, -8):
        if epw % k == 0:
            return k
    raise ValueError(f"per-worker edge count {epw} has no aligned chunk size")


def _padded_rows(N):
    CH = _chunk_size(N)
    ncopy = -(-(N // CH) // _NS)
    return _NS * ncopy * CH


@functools.lru_cache(maxsize=None)
def _make_agg(N, E, W):
    epw = E // _NW
    K = _chunk_size(epw, 80)
    nchunks = epw // K
    CH = _chunk_size(N)
    ncopy = -(-(N // CH) // _NS)
    Np = _NS * ncopy * CH

    mesh = plsc.VectorSubcoreMesh(core_axis_name="c", subcore_axis_name="s")

    @functools.partial(
        pl.kernel, mesh=mesh,
        out_type=(jax.ShapeDtypeStruct((_NC, Np, W), jnp.float32),),
        scratch_types=(
            pltpu.VMEM((K,), jnp.int32),
            pltpu.VMEM((K,), jnp.int32),
            pltpu.VMEM((K, W), jnp.float32),
            pltpu.VMEM((CH, W), jnp.float32),
            pltpu.VMEM_SHARED((Np, W), jnp.float32),
            pltpu.SemaphoreType.DMA,
        ),
        name=f"sage_agg_w{W}")
    def agg(table, src, dst, zrows, part, src_v, dst_v, rows_v, zbuf,
            agg_sh, sem):
        c = lax.axis_index("c")
        s = lax.axis_index("s")
        wid = s * _NC + c
        row0 = s * ncopy * CH

        pltpu.sync_copy(zrows, zbuf)
        for j in range(ncopy):
            pltpu.sync_copy(zbuf, agg_sh.at[pl.ds(row0 + j * CH, CH)])
        plsc.subcore_barrier()

        def chunk(i, carry):
            base = wid * epw + i * K
            pltpu.sync_copy(src.at[pl.ds(base, K)], src_v)
            pltpu.sync_copy(dst.at[pl.ds(base, K)], dst_v)
            pltpu.async_copy(table.at[src_v], rows_v, sem).wait()
            pltpu.sync_copy(rows_v, agg_sh.at[dst_v], add=True)
            return carry

        lax.fori_loop(0, nchunks, chunk, 0)
        plsc.subcore_barrier()

        for j in range(ncopy):
            base = row0 + j * CH
            pltpu.sync_copy(agg_sh.at[pl.ds(base, CH)], zbuf)
            pltpu.sync_copy(zbuf, part.at[c, pl.ds(base, CH)])

    return agg


@functools.lru_cache(maxsize=None)
def _make_deg(N, E, W):
    epw = E // _NW
    K = _chunk_size(epw, 80)
    nchunks = epw // K
    CH = _chunk_size(N)
    ncopy = -(-(N // CH) // _NS)
    Np = _NS * ncopy * CH

    mesh = plsc.VectorSubcoreMesh(core_axis_name="c", subcore_axis_name="s")

    @functools.partial(
        pl.kernel, mesh=mesh,
        out_type=(jax.ShapeDtypeStruct((_NC, Np, W), jnp.float32),),
        scratch_types=(
            pltpu.VMEM((K,), jnp.int32),
            pltpu.VMEM((K, W), jnp.float32),
            pltpu.VMEM((CH, W), jnp.float32),
            pltpu.VMEM_SHARED((Np, W), jnp.float32),
        ),
        name="sage_deg")
    def deg(dst, zrows, ones_h, dpart, dst_v, ones_v, zbuf, deg_sh):
        c = lax.axis_index("c")
        s = lax.axis_index("s")
        wid = s * _NC + c
        row0 = s * ncopy * CH

        pltpu.sync_copy(zrows, zbuf)
        pltpu.sync_copy(ones_h, ones_v)
        for j in range(ncopy):
            pltpu.sync_copy(zbuf, deg_sh.at[pl.ds(row0 + j * CH, CH)])
        plsc.subcore_barrier()

        def chunk(i, carry):
            base = wid * epw + i * K
            pltpu.sync_copy(dst.at[pl.ds(base, K)], dst_v)
            pltpu.sync_copy(ones_v, deg_sh.at[dst_v], add=True)
            return carry

        lax.fori_loop(0, nchunks, chunk, 0)
        plsc.subcore_barrier()

        for j in range(ncopy):
            base = row0 + j * CH
            pltpu.sync_copy(deg_sh.at[pl.ds(base, CH)], zbuf)
            pltpu.sync_copy(zbuf, dpart.at[c, pl.ds(base, CH)])

    return deg


def _tc_layer(h, part, deg16, w_s, b_s, w_n, b_n, gamma, beta):
    N, F = h.shape
    H = w_s.shape[1]

    def body(h_ref, p_ref, d_ref, ws_ref, bs_ref, wn_ref, bn_ref,
             g_ref, be_ref, y_ref):
        deg = d_ref[0, :N] + d_ref[1, :N]
        inv = 1.0 / jnp.maximum(deg[:, 0:1], 1.0)
        a = (p_ref[0, :N, :F] + p_ref[1, :N, :F]) * inv
        z = (jnp.dot(h_ref[...], ws_ref[...], preferred_element_type=jnp.float32)
             + jnp.dot(a, wn_ref[...], preferred_element_type=jnp.float32)
             + bs_ref[...] + bn_ref[...])
        mu = jnp.mean(z, axis=0, keepdims=True)
        var = jnp.mean((z - mu) ** 2, axis=0, keepdims=True)
        yn = (z - mu) * lax.rsqrt(var + _EPS) * g_ref[...] + be_ref[...]
        y_ref[...] = jnp.maximum(yn, 0.0)

    return pl.pallas_call(
        body,
        out_shape=jax.ShapeDtypeStruct((N, H), jnp.float32),
        name="sage_dense_bn_relu",
    )(h, part, deg16, w_s, b_s.reshape(1, -1), w_n, b_n.reshape(1, -1),
      gamma.reshape(1, -1), beta.reshape(1, -1))


def _tc_layer_final(h, part, deg16, w_s, b_s, w_n, b_n):
    N, F = h.shape
    C = w_s.shape[1]

    def body(h_ref, p_ref, d_ref, ws_ref, bs_ref, wn_ref, bn_ref, y_ref):
        deg = d_ref[0, :N] + d_ref[1, :N]
        inv = 1.0 / jnp.maximum(deg[:, 0:1], 1.0)
        a = (p_ref[0, :N, :F] + p_ref[1, :N, :F]) * inv
        y_ref[...] = (
            jnp.dot(h_ref[...], ws_ref[...], preferred_element_type=jnp.float32)
            + jnp.dot(a, wn_ref[...], preferred_element_type=jnp.float32)
            + bs_ref[...] + bn_ref[...])

    return pl.pallas_call(
        body,
        out_shape=jax.ShapeDtypeStruct((N, C), jnp.float32),
        name="sage_dense_final",
    )(h, part, deg16, w_s, b_s.reshape(1, -1), w_n, b_n.reshape(1, -1))


def kernel(x, edge_index,
           W_self0, b_self0, W_neigh0, b_neigh0, gamma0, beta0,
           W_self1, b_self1, W_neigh1, b_neigh1, gamma1, beta1,
           W_self2, b_self2, W_neigh2, b_neigh2):
    N, D = x.shape
    E = edge_index.shape[1]
    src = edge_index[0]
    dst = edge_index[1]

    CH = _chunk_size(N)
    epw = E // _NW
    K = _chunk_size(epw, 80)

    zrows = jnp.zeros((CH, D), jnp.float32)
    ones_h = jnp.ones((K, D), jnp.float32)

    (deg16,) = _make_deg(N, E, D)(dst, zrows, ones_h)
    (part0,) = _make_agg(N, E, D)(x, src, dst, zrows)
    h1 = _tc_layer(x, part0, deg16, W_self0, b_self0, W_neigh0, b_neigh0,
                   gamma0, beta0)
    (part1,) = _make_agg(N, E, D)(h1, src, dst, zrows)
    h2 = _tc_layer(h1, part1, deg16, W_self1, b_self1, W_neigh1, b_neigh1,
                   gamma1, beta1)
    (part2,) = _make_agg(N, E, D)(h2, src, dst, zrows)
    return _tc_layer_final(h2, part2, deg16, W_self2, b_self2,
                           W_neigh2, b_neigh2)


# speedup vs baseline: 4.9819x; 1.0525x over previous
import functools

import jax
import jax.numpy as jnp
from jax import lax
from jax.experimental import pallas as pl
from jax.experimental.pallas import tpu as pltpu
from jax.experimental.pallas import tpu_sc as plsc

_EPS = 1e-5
_NC = 2
_NS = 16
_NW = _NC * _NS


def _chunk_size(epw, cap=128):
    for k in range(cap, 0, -8):
        if epw % k == 0:
            return k
    raise ValueError(f"per-worker edge count {epw} has no aligned chunk size")


def _padded_rows(N):
    CH = _chunk_size(N)
    ncopy = -(-(N // CH) // _NS)
    return _NS * ncopy * CH


@functools.lru_cache(maxsize=None)
def _make_agg(N, E, W):
    epw = E // _NW
    K = _chunk_size(epw, 40)
    nchunks = epw // K
    assert nchunks % 2 == 0 and nchunks >= 4
    CH = _chunk_size(N)
    ncopy = -(-(N // CH) // _NS)
    Np = _NS * ncopy * CH

    mesh = plsc.VectorSubcoreMesh(core_axis_name="c", subcore_axis_name="s")

    @functools.partial(
        pl.kernel, mesh=mesh,
        out_type=(jax.ShapeDtypeStruct((_NC, Np, W), jnp.float32),),
        scratch_types=(
            pltpu.VMEM((K,), jnp.int32),
            pltpu.VMEM((K,), jnp.int32),
            pltpu.VMEM((K,), jnp.int32),
            pltpu.VMEM((K,), jnp.int32),
            pltpu.VMEM((K, W), jnp.float32),
            pltpu.VMEM((K, W), jnp.float32),
            pltpu.VMEM((CH, W), jnp.float32),
            pltpu.VMEM_SHARED((Np, W), jnp.float32),
            pltpu.SemaphoreType.DMA,
            pltpu.SemaphoreType.DMA,
        ),
        name=f"sage_agg_w{W}")
    def agg(table, src, dst, zrows, part, src_a, dst_a, src_b, dst_b,
            rows_a, rows_b, zbuf, agg_sh, sem_a, sem_b):
        c = lax.axis_index("c")
        s = lax.axis_index("s")
        wid = s * _NC + c
        row0 = s * ncopy * CH
        e0 = wid * epw

        pltpu.sync_copy(zrows, zbuf)
        for j in range(ncopy):
            pltpu.sync_copy(zbuf, agg_sh.at[pl.ds(row0 + j * CH, CH)])
        plsc.subcore_barrier()

        pltpu.sync_copy(src.at[pl.ds(e0, K)], src_a)
        pltpu.sync_copy(dst.at[pl.ds(e0, K)], dst_a)
        pltpu.async_copy(table.at[src_a], rows_a, sem_a)

        def wait_a():
            pltpu.make_async_copy(table.at[src_a], rows_a, sem_a).wait()

        def wait_b():
            pltpu.make_async_copy(table.at[src_b], rows_b, sem_b).wait()

        def pair(g, carry):
            ba = e0 + (2 * g + 1) * K
            pltpu.sync_copy(src.at[pl.ds(ba, K)], src_b)
            pltpu.sync_copy(dst.at[pl.ds(ba, K)], dst_b)
            pltpu.async_copy(table.at[src_b], rows_b, sem_b)
            wait_a()
            pltpu.sync_copy(rows_a, agg_sh.at[dst_a], add=True)
            bb = e0 + (2 * g + 2) * K
            pltpu.sync_copy(src.at[pl.ds(bb, K)], src_a)
            pltpu.sync_copy(dst.at[pl.ds(bb, K)], dst_a)
            pltpu.async_copy(table.at[src_a], rows_a, sem_a)
            wait_b()
            pltpu.sync_copy(rows_b, agg_sh.at[dst_b], add=True)
            return carry

        lax.fori_loop(0, nchunks // 2 - 1, pair, 0)

        bl = e0 + (nchunks - 1) * K
        pltpu.sync_copy(src.at[pl.ds(bl, K)], src_b)
        pltpu.sync_copy(dst.at[pl.ds(bl, K)], dst_b)
        pltpu.async_copy(table.at[src_b], rows_b, sem_b)
        wait_a()
        pltpu.sync_copy(rows_a, agg_sh.at[dst_a], add=True)
        wait_b()
        pltpu.sync_copy(rows_b, agg_sh.at[dst_b], add=True)
        plsc.subcore_barrier()

        for j in range(ncopy):
            base = row0 + j * CH
            pltpu.sync_copy(agg_sh.at[pl.ds(base, CH)], zbuf)
            pltpu.sync_copy(zbuf, part.at[c, pl.ds(base, CH)])

    return agg


@functools.lru_cache(maxsize=None)
def _make_deg(N, E, W):
    epw = E // _NW
    K = _chunk_size(epw, 80)
    nchunks = epw // K
    CH = _chunk_size(N)
    ncopy = -(-(N // CH) // _NS)
    Np = _NS * ncopy * CH

    mesh = plsc.VectorSubcoreMesh(core_axis_name="c", subcore_axis_name="s")

    @functools.partial(
        pl.kernel, mesh=mesh,
        out_type=(jax.ShapeDtypeStruct((_NC, Np, W), jnp.float32),),
        scratch_types=(
            pltpu.VMEM((K,), jnp.int32),
            pltpu.VMEM((K, W), jnp.float32),
            pltpu.VMEM((CH, W), jnp.float32),
            pltpu.VMEM_SHARED((Np, W), jnp.float32),
        ),
        name="sage_deg")
    def deg(dst, zrows, ones_h, dpart, dst_v, ones_v, zbuf, deg_sh):
        c = lax.axis_index("c")
        s = lax.axis_index("s")
        wid = s * _NC + c
        row0 = s * ncopy * CH

        pltpu.sync_copy(zrows, zbuf)
        pltpu.sync_copy(ones_h, ones_v)
        for j in range(ncopy):
            pltpu.sync_copy(zbuf, deg_sh.at[pl.ds(row0 + j * CH, CH)])
        plsc.subcore_barrier()

        def chunk(i, carry):
            base = wid * epw + i * K
            pltpu.sync_copy(dst.at[pl.ds(base, K)], dst_v)
            pltpu.sync_copy(ones_v, deg_sh.at[dst_v], add=True)
            return carry

        lax.fori_loop(0, nchunks, chunk, 0)
        plsc.subcore_barrier()

        for j in range(ncopy):
            base = row0 + j * CH
            pltpu.sync_copy(deg_sh.at[pl.ds(base, CH)], zbuf)
            pltpu.sync_copy(zbuf, dpart.at[c, pl.ds(base, CH)])

    return deg


def _tc_layer(h, part, deg16, w_s, b_s, w_n, b_n, gamma, beta):
    N, F = h.shape
    H = w_s.shape[1]

    def body(h_ref, p_ref, d_ref, ws_ref, bs_ref, wn_ref, bn_ref,
             g_ref, be_ref, y_ref):
        deg = d_ref[0, :N] + d_ref[1, :N]
        inv = 1.0 / jnp.maximum(deg[:, 0:1], 1.0)
        a = (p_ref[0, :N, :F] + p_ref[1, :N, :F]) * inv
        z = (jnp.dot(h_ref[...], ws_ref[...], preferred_element_type=jnp.float32)
             + jnp.dot(a, wn_ref[...], preferred_element_type=jnp.float32)
             + bs_ref[...] + bn_ref[...])
        mu = jnp.mean(z, axis=0, keepdims=True)
        var = jnp.mean((z - mu) ** 2, axis=0, keepdims=True)
        yn = (z - mu) * lax.rsqrt(var + _EPS) * g_ref[...] + be_ref[...]
        y_ref[...] = jnp.maximum(yn, 0.0)

    return pl.pallas_call(
        body,
        out_shape=jax.ShapeDtypeStruct((N, H), jnp.float32),
        name="sage_dense_bn_relu",
    )(h, part, deg16, w_s, b_s.reshape(1, -1), w_n, b_n.reshape(1, -1),
      gamma.reshape(1, -1), beta.reshape(1, -1))


def _tc_layer_final(h, part, deg16, w_s, b_s, w_n, b_n):
    N, F = h.shape
    C = w_s.shape[1]

    def body(h_ref, p_ref, d_ref, ws_ref, bs_ref, wn_ref, bn_ref, y_ref):
        deg = d_ref[0, :N] + d_ref[1, :N]
        inv = 1.0 / jnp.maximum(deg[:, 0:1], 1.0)
        a = (p_ref[0, :N, :F] + p_ref[1, :N, :F]) * inv
        y_ref[...] = (
            jnp.dot(h_ref[...], ws_ref[...], preferred_element_type=jnp.float32)
            + jnp.dot(a, wn_ref[...], preferred_element_type=jnp.float32)
            + bs_ref[...] + bn_ref[...])

    return pl.pallas_call(
        body,
        out_shape=jax.ShapeDtypeStruct((N, C), jnp.float32),
        name="sage_dense_final",
    )(h, part, deg16, w_s, b_s.reshape(1, -1), w_n, b_n.reshape(1, -1))


def kernel(x, edge_index,
           W_self0, b_self0, W_neigh0, b_neigh0, gamma0, beta0,
           W_self1, b_self1, W_neigh1, b_neigh1, gamma1, beta1,
           W_self2, b_self2, W_neigh2, b_neigh2):
    N, D = x.shape
    E = edge_index.shape[1]
    src = edge_index[0]
    dst = edge_index[1]

    CH = _chunk_size(N)
    epw = E // _NW
    K = _chunk_size(epw, 80)

    zrows = jnp.zeros((CH, D), jnp.float32)
    ones_h = jnp.ones((K, D), jnp.float32)

    (deg16,) = _make_deg(N, E, D)(dst, zrows, ones_h)
    (part0,) = _make_agg(N, E, D)(x, src, dst, zrows)
    h1 = _tc_layer(x, part0, deg16, W_self0, b_self0, W_neigh0, b_neigh0,
                   gamma0, beta0)
    (part1,) = _make_agg(N, E, D)(h1, src, dst, zrows)
    h2 = _tc_layer(h1, part1, deg16, W_self1, b_self1, W_neigh1, b_neigh1,
                   gamma1, beta1)
    (part2,) = _make_agg(N, E, D)(h2, src, dst, zrows)
    return _tc_layer_final(h2, part2, deg16, W_self2, b_self2,
                           W_neigh2, b_neigh2)


# speedup vs baseline: 6.8962x; 1.3842x over previous
import functools

import jax
import jax.numpy as jnp
from jax import lax
from jax.experimental import pallas as pl
from jax.experimental.pallas import tpu as pltpu
from jax.experimental.pallas import tpu_sc as plsc

_EPS = 1e-5
_NC = 2
_NS = 16
_NW = _NC * _NS


def _chunk_size(epw, cap=128):
    for k in range(cap, 0, -8):
        if epw % k == 0:
            return k
    raise ValueError(f"per-worker edge count {epw} has no aligned chunk size")


def _padded_rows(N):
    CH = _chunk_size(N)
    ncopy = -(-(N // CH) // _NS)
    return _NS * ncopy * CH


@functools.lru_cache(maxsize=None)
def _make_agg(N, E, W):
    epw = E // _NW
    K = _chunk_size(epw, 80)
    nchunks = epw // K
    CH = _chunk_size(N)
    ncopy = -(-(N // CH) // _NS)
    Np = _NS * ncopy * CH

    mesh = plsc.VectorSubcoreMesh(core_axis_name="c", subcore_axis_name="s")

    @functools.partial(
        pl.kernel, mesh=mesh,
        out_type=(jax.ShapeDtypeStruct((_NC, Np, W), jnp.float32),),
        scratch_types=(
            pltpu.VMEM((epw,), jnp.int32),
            pltpu.VMEM((nchunks, K), jnp.int32),
            pltpu.VMEM((K, W), jnp.float32),
            pltpu.VMEM((CH, W), jnp.float32),
            pltpu.VMEM_SHARED((Np, W), jnp.float32),
            pltpu.SemaphoreType.DMA,
        ),
        name=f"sage_agg_w{W}")
    def agg(table, src3, dst3, zrows, part, src_all, dst_all,
            rows_a, zbuf, agg_sh, sem_a):
        c = lax.axis_index("c")
        s = lax.axis_index("s")
        wid = s * _NC + c
        row0 = s * ncopy * CH

        pltpu.sync_copy(src3.at[wid], src_all)
        pltpu.sync_copy(dst3.at[wid], dst_all)

        pltpu.sync_copy(zrows, zbuf)
        for j in range(ncopy):
            pltpu.sync_copy(zbuf, agg_sh.at[pl.ds(row0 + j * CH, CH)])
        plsc.subcore_barrier()

        def chunk(i, carry):
            pltpu.async_copy(table.at[src_all.at[pl.ds(i * K, K)]], rows_a,
                             sem_a).wait()
            pltpu.sync_copy(rows_a, agg_sh.at[dst_all.at[i]], add=True)
            return carry

        lax.fori_loop(0, nchunks, chunk, 0)
        plsc.subcore_barrier()

        for j in range(ncopy):
            base = row0 + j * CH
            pltpu.sync_copy(agg_sh.at[pl.ds(base, CH)], zbuf)
            pltpu.sync_copy(zbuf, part.at[c, pl.ds(base, CH)])

    return agg


@functools.lru_cache(maxsize=None)
def _make_deg(N, E, W):
    epw = E // _NW
    K = _chunk_size(epw, 80)
    nchunks = epw // K
    CH = _chunk_size(N)
    ncopy = -(-(N // CH) // _NS)
    Np = _NS * ncopy * CH

    mesh = plsc.VectorSubcoreMesh(core_axis_name="c", subcore_axis_name="s")

    @functools.partial(
        pl.kernel, mesh=mesh,
        out_type=(jax.ShapeDtypeStruct((_NC, Np, W), jnp.float32),),
        scratch_types=(
            pltpu.VMEM((nchunks, K), jnp.int32),
            pltpu.VMEM((K, W), jnp.float32),
            pltpu.VMEM((CH, W), jnp.float32),
            pltpu.VMEM_SHARED((Np, W), jnp.float32),
        ),
        name="sage_deg")
    def deg(dst3, zrows, ones_h, dpart, dst_all, ones_v, zbuf, deg_sh):
        c = lax.axis_index("c")
        s = lax.axis_index("s")
        wid = s * _NC + c
        row0 = s * ncopy * CH

        pltpu.sync_copy(dst3.at[wid], dst_all)
        pltpu.sync_copy(zrows, zbuf)
        pltpu.sync_copy(ones_h, ones_v)
        for j in range(ncopy):
            pltpu.sync_copy(zbuf, deg_sh.at[pl.ds(row0 + j * CH, CH)])
        plsc.subcore_barrier()

        def chunk(i, carry):
            pltpu.sync_copy(ones_v, deg_sh.at[dst_all.at[i]], add=True)
            return carry

        lax.fori_loop(0, nchunks, chunk, 0)
        plsc.subcore_barrier()

        for j in range(ncopy):
            base = row0 + j * CH
            pltpu.sync_copy(deg_sh.at[pl.ds(base, CH)], zbuf)
            pltpu.sync_copy(zbuf, dpart.at[c, pl.ds(base, CH)])

    return deg


def _tc_layer(h, part, deg16, w_s, b_s, w_n, b_n, gamma, beta):
    N, F = h.shape
    H = w_s.shape[1]

    def body(h_ref, p_ref, d_ref, ws_ref, bs_ref, wn_ref, bn_ref,
             g_ref, be_ref, y_ref):
        deg = d_ref[0, :N] + d_ref[1, :N]
        inv = 1.0 / jnp.maximum(deg[:, 0:1], 1.0)
        a = (p_ref[0, :N, :F] + p_ref[1, :N, :F]) * inv
        z = (jnp.dot(h_ref[...], ws_ref[...], preferred_element_type=jnp.float32)
             + jnp.dot(a, wn_ref[...], preferred_element_type=jnp.float32)
             + bs_ref[...] + bn_ref[...])
        mu = jnp.mean(z, axis=0, keepdims=True)
        var = jnp.mean((z - mu) ** 2, axis=0, keepdims=True)
        yn = (z - mu) * lax.rsqrt(var + _EPS) * g_ref[...] + be_ref[...]
        y_ref[...] = jnp.maximum(yn, 0.0)

    return pl.pallas_call(
        body,
        out_shape=jax.ShapeDtypeStruct((N, H), jnp.float32),
        name="sage_dense_bn_relu",
    )(h, part, deg16, w_s, b_s.reshape(1, -1), w_n, b_n.reshape(1, -1),
      gamma.reshape(1, -1), beta.reshape(1, -1))


def _tc_layer_final(h, part, deg16, w_s, b_s, w_n, b_n):
    N, F = h.shape
    C = w_s.shape[1]

    def body(h_ref, p_ref, d_ref, ws_ref, bs_ref, wn_ref, bn_ref, y_ref):
        deg = d_ref[0, :N] + d_ref[1, :N]
        inv = 1.0 / jnp.maximum(deg[:, 0:1], 1.0)
        a = (p_ref[0, :N, :F] + p_ref[1, :N, :F]) * inv
        y_ref[...] = (
            jnp.dot(h_ref[...], ws_ref[...], preferred_element_type=jnp.float32)
            + jnp.dot(a, wn_ref[...], preferred_element_type=jnp.float32)
            + bs_ref[...] + bn_ref[...])

    return pl.pallas_call(
        body,
        out_shape=jax.ShapeDtypeStruct((N, C), jnp.float32),
        name="sage_dense_final",
    )(h, part, deg16, w_s, b_s.reshape(1, -1), w_n, b_n.reshape(1, -1))


def kernel(x, edge_index,
           W_self0, b_self0, W_neigh0, b_neigh0, gamma0, beta0,
           W_self1, b_self1, W_neigh1, b_neigh1, gamma1, beta1,
           W_self2, b_self2, W_neigh2, b_neigh2):
    N, D = x.shape
    E = edge_index.shape[1]
    src = edge_index[0]
    dst = edge_index[1]

    CH = _chunk_size(N)
    epw = E // _NW
    K_deg = _chunk_size(epw, 80)
    K_agg = _chunk_size(epw, 80)

    zrows = jnp.zeros((CH, D), jnp.float32)
    ones_h = jnp.ones((K_deg, D), jnp.float32)
    src3 = src.reshape(_NW, epw)
    dst3 = dst.reshape(_NW, epw // K_agg, K_agg)
    dst3d = dst.reshape(_NW, epw // K_deg, K_deg)

    (deg16,) = _make_deg(N, E, D)(dst3d, zrows, ones_h)
    (part0,) = _make_agg(N, E, D)(x, src3, dst3, zrows)
    h1 = _tc_layer(x, part0, deg16, W_self0, b_self0, W_neigh0, b_neigh0,
                   gamma0, beta0)
    (part1,) = _make_agg(N, E, D)(h1, src3, dst3, zrows)
    h2 = _tc_layer(h1, part1, deg16, W_self1, b_self1, W_neigh1, b_neigh1,
                   gamma1, beta1)
    (part2,) = _make_agg(N, E, D)(h2, src3, dst3, zrows)
    return _tc_layer_final(h2, part2, deg16, W_self2, b_self2,
                           W_neigh2, b_neigh2)
